# baseline (speedup 1.0000x reference)
import jax
import jax.numpy as jnp
from jax.experimental import pallas as pl

B = 32; G = 10000; NGO = 2048; NKE = 256
NN = 8; NL = 20; NT = 32; DC = 256
EPS = 1e-5


def _linear(x, p):
    return x @ p["w"] + p["b"]


def _bn3(x, g, b):
    return x / jnp.sqrt(1.0 + EPS) * g[None, :, None] + b[None, :, None]


def _bn2(x, g, b):
    return x / jnp.sqrt(1.0 + EPS) * g[None, :] + b[None, :]


def _vnn_fwd(x, edge_index, p, num_nodes):
    src = edge_index[0]
    dst = edge_index[1]
    xg = x[:, src, :]
    h = jnp.einsum('bei,eio->beo', xg, p["weight"]) + p["bias"][None]
    h = jnp.tanh(h)
    seg = jax.vmap(lambda hb: jax.ops.segment_sum(hb, dst, num_segments=num_nodes))(h)
    cnt = jax.ops.segment_sum(jnp.ones((dst.shape[0],), x.dtype), dst, num_segments=num_nodes)
    mean = seg / jnp.maximum(cnt, 1.0)[None, :, None]
    state = _linear(mean, p["state"]).squeeze(-1)
    out = _bn3(mean, p["bn_g"], p["bn_b"])
    return state, out


def _mlp_fwd(x, p):
    h = jax.nn.sigmoid(_linear(x, p["l0"]))
    h = _bn2(h, p["bn0_g"], p["bn0_b"])
    h = jax.nn.sigmoid(_linear(h, p["l1"]))
    h = _bn2(h, p["bn1_g"], p["bn1_b"])
    h = jax.nn.sigmoid(_linear(h, p["l2"]))
    h = _bn2(h, p["bn2_g"], p["bn2_b"])
    return _linear(h, p["l3"])


def _final_linear_kernel(x_ref, w_ref, b_ref, o_ref):
    o_ref[...] = x_ref[...] @ w_ref[...] + b_ref[...]


def kernel(gene, edge_gene_go, edge_go_ke, edge_ke_ke, tissue, c, params):
    g = jnp.tanh(_linear(gene[..., None], params["gene_layer"]))
    g = _bn3(g, params["gene_bn_g"], params["gene_bn_b"])
    gene_state = _linear(g, params["gene_state"]).squeeze(-1)
    go_state, go = _vnn_fwd(g, edge_gene_go, params["gene2go"], NGO)
    _, ke = _vnn_fwd(go, edge_go_ke, params["go2ke"], NKE)
    ke_state = None
    for p in params["ke2ke"]:
        ke_state, ke = _vnn_fwd(ke, edge_ke_ke, p, NKE)
    bio_node_state = jnp.concatenate([gene_state, go_state, ke_state], axis=-1)
    ke_s = _linear(ke, params["ke_layer"]).squeeze(-1)
    bio_pred = _mlp_fwd(ke_s[:, tissue], params["bio"])
    drug_pred = _mlp_fwd(c, params["drug"])
    pp = params["predict"]
    cat = jnp.concatenate([bio_pred, drug_pred], axis=-1)
    result = pl.pallas_call(
        _final_linear_kernel,
        out_shape=jax.ShapeDtypeStruct((B, NL), jnp.float32),
    )(cat, pp["w"], jnp.broadcast_to(pp["b"], (B, NL)))
    return (_linear(bio_node_state, params["reshaper"]), result)


# speedup vs baseline: 10.7263x; 10.7263x over previous
import functools

import jax
import jax.numpy as jnp
from jax import lax
from jax.experimental import pallas as pl
from jax.experimental.pallas import tpu as pltpu
from jax.experimental.pallas import tpu_sc as plsc

B = 32; G = 10000; NGO = 2048; NKE = 256
NN = 8; NL = 20; NT = 32; DC = 256
EPS = 1e-5
INV_SQRT = 1.0 / (1.0 + EPS) ** 0.5

NC = 2
NS = 16
NW = NC * NS
K = 64
ACCW = 272


def _tanh16(z):
    return 1.0 - 2.0 / (1.0 + jnp.exp(2.0 * z))


def _edge_math(xg, wv, e):
    xs = [xg[e, 16 * t:16 * (t + 1)] for t in range(16)]
    wvecs = [wv[e, 16 * q:16 * (q + 1)] for q in range(4)]
    bvec = wv[e, 64:80]
    out = []
    for o in range(8):
        bo = bvec[o]
        h0 = jnp.full((16,), bo)
        h1 = jnp.full((16,), bo)
        for i in range(8):
            kk = i * 8 + o
            w = wvecs[kk // 16][kk % 16]
            h0 = h0 + xs[2 * i] * w
            h1 = h1 + xs[2 * i + 1] * w
        out.append(_tanh16(h0))
        out.append(_tanh16(h1))
    return out


@functools.lru_cache(maxsize=None)
def _make_vnn_acc(e_pad, nsrc, ndst_pad, nblk):
    mesh = plsc.VectorSubcoreMesh(core_axis_name="c", subcore_axis_name="s")
    acc_rows = ndst_pad + ndst_pad // 16
    nidx = nblk * 2 * K

    @functools.partial(
        pl.kernel,
        mesh=mesh,
        out_type=jax.ShapeDtypeStruct((NW, acc_rows, 256), jnp.float32),
        scratch_types=[
            pltpu.VMEM((nidx + 16,), jnp.int32),
            pltpu.VMEM((K, 256), jnp.float32),
            pltpu.VMEM((K, 256), jnp.float32),
            pltpu.VMEM((K, 80), jnp.float32),
            pltpu.VMEM((K, 80), jnp.float32),
            pltpu.VMEM((acc_rows, 256), jnp.float32),
            pltpu.SemaphoreType.DMA, pltpu.SemaphoreType.DMA,
            pltpu.SemaphoreType.DMA, pltpu.SemaphoreType.DMA,
        ],
    )
    def vnn(x_hbm, wb_hbm, sd_hbm, out_hbm,
            sdv, xg0, xg1, wv0, wv1, acc, g0, g1, w0, w1):
        c = lax.axis_index("c")
        s = lax.axis_index("s")
        wid = s * NC + c
        xgs, wvs, gsems, wsems = [xg0, xg1], [wv0, wv1], [g0, g1], [w0, w1]

        pltpu.sync_copy(sd_hbm.at[pl.ds(wid * nidx, nidx)],
                        sdv.at[pl.ds(0, nidx)])

        zero16 = jnp.zeros((16,), jnp.float32)

        def zrow(r, _):
            for q in range(16):
                acc[r, q * 16:(q + 1) * 16] = zero16
            return 0

        lax.fori_loop(0, acc_rows, zrow, 0)
        cntvec = jnp.where(lax.iota(jnp.int32, 16) == 0, 1.0, 0.0)

        for j in range(2):
            pltpu.async_copy(x_hbm.at[sdv.at[pl.ds(j * 2 * K, K)]],
                             xgs[j], gsems[j])
            pltpu.async_copy(wb_hbm.at[pl.ds(wid * nblk * K + j * K, K)],
                             wvs[j], wsems[j])

        def pair(i, _):
            for j in range(2):
                blk = 2 * i + j
                pltpu.make_async_copy(x_hbm.at[pl.ds(0, K)],
                                      xgs[j], gsems[j]).wait()
                pltpu.make_async_copy(wb_hbm.at[pl.ds(0, K)],
                                      wvs[j], wsems[j]).wait()
                dbase = blk * 2 * K + K

                def edge(e, _, _j=j, _dbase=dbase):
                    hs = _edge_math(xgs[_j], wvs[_j], e)
                    d = sdv[pl.ds(_dbase + e, 16)][0]
                    for t in range(16):
                        plsc.addupdate(acc.at[d, pl.ds(16 * t, 16)], hs[t])
                    crow = ndst_pad + lax.shift_right_logical(d, 4)
                    ccol = (d & 15) * 16
                    plsc.addupdate(acc.at[crow, pl.ds(ccol, 16)], cntvec)
                    return 0

                lax.fori_loop(0, K, edge, 0, unroll=2)
                nxt = blk + 2

                @pl.when(nxt < nblk)
                def _(_j=j, _nxt=nxt):
                    pltpu.async_copy(x_hbm.at[sdv.at[pl.ds(_nxt * 2 * K, K)]],
                                     xgs[_j], gsems[_j])
                    pltpu.async_copy(
                        wb_hbm.at[pl.ds(wid * nblk * K + _nxt * K, K)],
                        wvs[_j], wsems[_j])
            return 0

        lax.fori_loop(0, nblk // 2, pair, 0)
        pltpu.sync_copy(acc, out_hbm.at[wid])

    return vnn


@functools.lru_cache(maxsize=None)
def _make_vnn_edges(e_pad, nsrc, nblk):
    mesh = plsc.VectorSubcoreMesh(core_axis_name="c", subcore_axis_name="s")
    nidx = nblk * K

    @functools.partial(
        pl.kernel,
        mesh=mesh,
        out_type=jax.ShapeDtypeStruct((e_pad, ACCW), jnp.float32),
        scratch_types=[
            pltpu.VMEM((nidx,), jnp.int32),
            pltpu.VMEM((K, 256), jnp.float32),
            pltpu.VMEM((K, 256), jnp.float32),
            pltpu.VMEM((K, 80), jnp.float32),
            pltpu.VMEM((K, 80), jnp.float32),
            pltpu.VMEM((K, ACCW), jnp.float32),
            pltpu.VMEM((K, ACCW), jnp.float32),
            pltpu.SemaphoreType.DMA, pltpu.SemaphoreType.DMA,
            pltpu.SemaphoreType.DMA, pltpu.SemaphoreType.DMA,
            pltpu.SemaphoreType.DMA, pltpu.SemaphoreType.DMA,
        ],
    )
    def vnn(x_hbm, wb_hbm, src_hbm, out_hbm,
            srcv, xg0, xg1, wv0, wv1, hv0, hv1, g0, g1, w0, w1, o0, o1):
        c = lax.axis_index("c")
        s = lax.axis_index("s")
        wid = s * NC + c
        xgs, wvs, hvs = [xg0, xg1], [wv0, wv1], [hv0, hv1]
        gsems, wsems, osems = [g0, g1], [w0, w1], [o0, o1]

        pltpu.sync_copy(src_hbm.at[pl.ds(wid * nidx, nidx)], srcv)

        cntvec = jnp.where(lax.iota(jnp.int32, 16) == 0, 1.0, 0.0)

        def crow(e, _):
            hv0[e, 256:272] = cntvec
            hv1[e, 256:272] = cntvec
            return 0

        lax.fori_loop(0, K, crow, 0)

        for j in range(2):
            pltpu.async_copy(x_hbm.at[srcv.at[pl.ds(j * K, K)]],
                             xgs[j], gsems[j])
            pltpu.async_copy(wb_hbm.at[pl.ds(wid * nblk * K + j * K, K)],
                             wvs[j], wsems[j])

        def pair(i, _):
            for j in range(2):
                blk = 2 * i + j
                pltpu.make_async_copy(x_hbm.at[pl.ds(0, K)],
                                      xgs[j], gsems[j]).wait()
                pltpu.make_async_copy(wb_hbm.at[pl.ds(0, K)],
                                      wvs[j], wsems[j]).wait()

                @pl.when(blk >= 2)
                def _(_j=j):
                    pltpu.make_async_copy(hvs[_j],
                                          out_hbm.at[pl.ds(0, K)],
                                          osems[_j]).wait()

                def edge(e, _, _j=j):
                    hs = _edge_math(xgs[_j], wvs[_j], e)
                    for t in range(16):
                        hvs[_j][e, 16 * t:16 * (t + 1)] = hs[t]
                    return 0

                lax.fori_loop(0, K, edge, 0, unroll=2)
                base = wid * nblk * K + blk * K
                pltpu.async_copy(hvs[j], out_hbm.at[pl.ds(base, K)], osems[j])
                nxt = blk + 2

                @pl.when(nxt < nblk)
                def _(_j=j, _nxt=nxt):
                    pltpu.async_copy(x_hbm.at[srcv.at[pl.ds(_nxt * K, K)]],
                                     xgs[_j], gsems[_j])
                    pltpu.async_copy(
                        wb_hbm.at[pl.ds(wid * nblk * K + _nxt * K, K)],
                        wvs[_j], wsems[_j])
            return 0

        lax.fori_loop(0, nblk // 2, pair, 0)
        for j in range(2):
            pltpu.make_async_copy(hvs[j], out_hbm.at[pl.ds(0, K)],
                                  osems[j]).wait()

    return vnn


def _pad_up(n, m):
    return ((n + m - 1) // m) * m


def _prep_edges(edge_index, p, nsrc, ndst):
    E = edge_index.shape[1]
    e_pad = _pad_up(E, NW * K * 2)
    npad = e_pad - E
    src = edge_index[0].astype(jnp.int32)
    dst = edge_index[1].astype(jnp.int32)
    if npad:
        src = jnp.concatenate([src, jnp.arange(npad, dtype=jnp.int32) % nsrc])
        dst = jnp.concatenate([dst, jnp.full((npad,), ndst, jnp.int32)])
    wb = jnp.concatenate([p["weight"].reshape(E, 64),
                          p["bias"],
                          jnp.zeros((E, 8), jnp.float32)], axis=1)
    if npad:
        wb = jnp.concatenate([wb, jnp.zeros((npad, 80), jnp.float32)])
    return src, dst, wb, e_pad


def _epilogue(tot, cnt, p, ndst):
    cnt = jnp.maximum(cnt[:ndst, None], 1.0)
    mean_t = tot[:ndst, :256] / cnt
    sp = p["state"]
    state_t = sum(mean_t[:, 32 * o:32 * (o + 1)] * sp["w"][o, 0] for o in range(8))
    state_t = state_t + sp["b"][0]
    out_t = mean_t * INV_SQRT * p["bn_g"][:, None] + p["bn_b"][:, None]
    return out_t, state_t


def _vnn_layer_small(x_t, edge_index, p, nsrc, ndst):
    src, dst, wb, e_pad = _prep_edges(edge_index, p, nsrc, ndst)
    nblk = e_pad // (NW * K)
    ndst_pad = _pad_up(ndst + 1, 16)
    sd = jnp.concatenate([src.reshape(-1, K), dst.reshape(-1, K)],
                         axis=1).reshape(-1)
    vnn = _make_vnn_acc(e_pad, nsrc, ndst_pad, nblk)
    accs = vnn(x_t, wb, sd)
    tot = jnp.sum(accs, axis=0)
    cnt = tot[ndst_pad:].reshape(ndst_pad // 16, 16, 16)[:, :, 0].reshape(-1)
    return _epilogue(tot[:ndst_pad], cnt, p, ndst)


def _vnn_layer_big(x_t, edge_index, p, nsrc, ndst):
    src, dst, wb, e_pad = _prep_edges(edge_index, p, nsrc, ndst)
    nblk = e_pad // (NW * K)
    vnn = _make_vnn_edges(e_pad, nsrc, nblk)
    h = vnn(x_t, wb, src)
    tot = jax.ops.segment_sum(h, dst, num_segments=ndst + 1)
    return _epilogue(tot[:, :256], tot[:, 256], p, ndst)


def _linear(x, p):
    return x @ p["w"] + p["b"]


def _bn2(x, g, b):
    return x * INV_SQRT * g[None, :] + b[None, :]


def _mlp_fwd(x, p):
    h = jax.nn.sigmoid(_linear(x, p["l0"]))
    h = _bn2(h, p["bn0_g"], p["bn0_b"])
    h = jax.nn.sigmoid(_linear(h, p["l1"]))
    h = _bn2(h, p["bn1_g"], p["bn1_b"])
    h = jax.nn.sigmoid(_linear(h, p["l2"]))
    h = _bn2(h, p["bn2_g"], p["bn2_b"])
    return _linear(h, p["l3"])


def _final_linear_kernel(x_ref, w_ref, b_ref, o_ref):
    o_ref[...] = x_ref[...] @ w_ref[...] + b_ref[...]


def kernel(gene, edge_gene_go, edge_go_ke, edge_ke_ke, tissue, c, params):
    gl = params["gene_layer"]
    gene_t = gene.T
    g_pre = jnp.tanh(gene_t[:, None, :] * gl["w"][0][None, :, None]
                     + gl["b"][None, :, None])
    g_bn = (g_pre * INV_SQRT * params["gene_bn_g"][:, None, None]
            + params["gene_bn_b"][:, None, None])
    x0 = g_bn.reshape(G, 256)
    gs = params["gene_state"]
    gene_state_t = jnp.einsum('gib,i->gb', g_bn, gs["w"][:, 0]) + gs["b"][0]

    go_t, go_state_t = _vnn_layer_big(x0, edge_gene_go, params["gene2go"], G, NGO)
    ke_t, _ = _vnn_layer_small(go_t, edge_go_ke, params["go2ke"], NGO, NKE)
    ke_state_t = None
    for p in params["ke2ke"]:
        ke_t, ke_state_t = _vnn_layer_small(ke_t, edge_ke_ke, p, NKE, NKE)

    rp = params["reshaper"]
    out1 = (gene_state_t.T @ rp["w"][:G]
            + go_state_t.T @ rp["w"][G:G + NGO]
            + ke_state_t.T @ rp["w"][G + NGO:]
            + rp["b"])

    klp = params["ke_layer"]
    ke_s_t = sum(ke_t[:, 32 * o:32 * (o + 1)] * klp["w"][o, 0] for o in range(8))
    ke_s_t = ke_s_t + klp["b"][0]
    bio_in = ke_s_t[tissue].T
    bio_pred = _mlp_fwd(bio_in, params["bio"])
    drug_pred = _mlp_fwd(c, params["drug"])
    pp = params["predict"]
    cat = jnp.concatenate([bio_pred, drug_pred], axis=-1)
    result = pl.pallas_call(
        _final_linear_kernel,
        out_shape=jax.ShapeDtypeStruct((B, NL), jnp.float32),
    )(cat, pp["w"], jnp.broadcast_to(pp["b"], (B, NL)))
    return (out1, result)
